# baseline (speedup 1.0000x reference)
import functools

import jax
import jax.numpy as jnp
from jax import lax
from jax.experimental import pallas as pl
from jax.experimental.pallas import tpu as pltpu
from jax.experimental.pallas import tpu_sc as plsc

_LANES = 16
_NCORES = 2
_NSUB = 16
_NW = _NCORES * _NSUB
_NBUF = 8
_SUB = 5
_RED_UNROLL = 8


@functools.lru_cache(maxsize=None)
def _build(B, L, V, E):
    bpw = B // _NW
    ek = E // _LANES
    ssz = L // _SUB
    assert B % _NW == 0 and E % _LANES == 0
    assert L % _SUB == 0 and ssz % 8 == 0 and ssz % _RED_UNROLL == 0
    assert bpw % _NBUF == 0

    nfull = L // _LANES
    tail = L - nfull * _LANES

    mesh = plsc.VectorSubcoreMesh(core_axis_name="c", subcore_axis_name="s")

    @functools.partial(
        pl.kernel,
        mesh=mesh,
        compiler_params=pltpu.CompilerParams(
            use_tc_tiling_on_sc=False, needs_layout_passes=False
        ),
        out_type=jax.ShapeDtypeStruct((B, E), jnp.float32),
        scratch_types=[
            pltpu.VMEM((bpw, L), jnp.int32),
            pltpu.VMEM((_NBUF, ssz, E), jnp.float32),
            pltpu.VMEM((_NBUF, E), jnp.float32),
            pltpu.VMEM((bpw, E), jnp.float32),
        ] + [pltpu.SemaphoreType.DMA] * _NBUF,
    )
    def enc(x_hbm, table_hbm, out_hbm, x_v, rows_v, csum_v, out_v, *sems):
        wid = lax.axis_index("s") * _NCORES + lax.axis_index("c")
        base = wid * bpw
        pltpu.sync_copy(x_hbm.at[pl.ds(base, bpw)], x_v)

        zed = jnp.zeros((_LANES,), jnp.float32)

        def zero_body(r, c):
            for buf in range(_NBUF):
                for k in range(ek):
                    rows_v[buf, r, pl.ds(k * _LANES, _LANES)] = zed
            return c

        lax.fori_loop(0, ssz, zero_body, 0)
        for buf in range(_NBUF):
            for k in range(ek):
                csum_v[buf, pl.ds(k * _LANES, _LANES)] = zed

        def start_row(b, buf):
            for j in range(_SUB):
                pltpu.async_copy(
                    table_hbm.at[plsc.Indices(x_v.at[b, pl.ds(j * ssz, ssz)])],
                    rows_v.at[buf],
                    sems[buf],
                    add=True,
                )

        def wait_row(buf):
            for j in range(_SUB):
                pltpu.make_async_copy(
                    table_hbm.at[pl.ds(0, ssz)],
                    rows_v.at[buf],
                    sems[buf],
                ).wait()

        def reduce_row(buf):
            def red_body(i, accs):
                accs = list(accs)
                for u in range(_RED_UNROLL):
                    r = i * _RED_UNROLL + u
                    for k in range(ek):
                        accs[k] = accs[k] + rows_v[buf, r, pl.ds(k * _LANES, _LANES)]
                return tuple(accs)

            accs = lax.fori_loop(
                0, ssz // _RED_UNROLL, red_body, (zed,) * ek
            )
            out = []
            for k in range(ek):
                prev = csum_v[buf, pl.ds(k * _LANES, _LANES)]
                out.append(accs[k] - prev)
                csum_v[buf, pl.ds(k * _LANES, _LANES)] = accs[k]
            return out

        def finalize(b, accs):
            cnt = jnp.zeros((_LANES,), jnp.int32)
            for j in range(nfull):
                v = x_v[b, pl.ds(j * _LANES, _LANES)]
                cnt = cnt + plsc.all_reduce_population_count(v != 0)
            if tail:
                v = x_v[b, pl.ds(L - _LANES, _LANES)]
                m = (v != 0) & (lax.iota(jnp.int32, _LANES) >= (_LANES - tail))
                cnt = cnt + plsc.all_reduce_population_count(m)
            inv = 1.0 / jnp.maximum(cnt.astype(jnp.float32), 1.0)
            for k in range(ek):
                out_v[b, pl.ds(k * _LANES, _LANES)] = accs[k] * inv

        for j in range(_NBUF - 1):
            start_row(j, j)

        def outer(g, carry):
            b0 = g * _NBUF
            for u in range(_NBUF):
                b = b0 + u
                wait_row(u)
                nb = b + (_NBUF - 1)

                @pl.when(nb < bpw)
                def _():
                    start_row(nb, (u + _NBUF - 1) % _NBUF)

                finalize(b, reduce_row(u))
            return carry

        lax.fori_loop(0, bpw // _NBUF, outer, 0)
        pltpu.sync_copy(out_v, out_hbm.at[pl.ds(base, bpw)])

    return enc


def kernel(x, lengths, table):
    del lengths
    B, L = x.shape
    V, E = table.shape
    return _build(B, L, V, E)(x.astype(jnp.int32), table)
